# Initial kernel scaffold; baseline (speedup 1.0000x reference)
import jax
import jax.numpy as jnp
from jax.experimental import pallas as pl


def kernel(x, edge_index, W, b):
    raise NotImplementedError("write your pallas kernel here")


# speedup vs baseline: 38.1210x; 38.1210x over previous
import functools

import jax
import jax.numpy as jnp
from jax import lax
from jax.experimental import pallas as pl
from jax.experimental.pallas import tpu as pltpu
from jax.experimental.pallas import tpu_sc as plsc

N = 10000
D = 128
NC = 2
NS = 16
NW = NC * NS
NPAD = 10240
RZ = NPAD // NS
CH = 128
K = 80
EPT = K * CH
EPAD = NW * EPT
NB = 2
NI = 4

@functools.cache
def _mesh():
    return plsc.VectorSubcoreMesh(
        core_axis_name="c", subcore_axis_name="s", num_cores=NC, num_subcores=NS
    )


def _deg_body(dst_hbm, out_hbm, deg_sh, didx, ones, zbuf, isem):
    c = lax.axis_index("c")
    s = lax.axis_index("s")
    t = c * NS + s
    base = t * EPT

    @pl.loop(0, CH // 16)
    def _fill_ones(i):
        ones[pl.ds(i * 16, 16)] = jnp.full((16,), 1.0, jnp.float32)

    @pl.loop(0, RZ // 16)
    def _fill_zeros(i):
        zbuf[pl.ds(i * 16, 16)] = jnp.zeros((16,), jnp.float32)

    pltpu.sync_copy(zbuf, deg_sh.at[pl.ds(s * RZ, RZ)])

    cps = [
        pltpu.async_copy(dst_hbm.at[pl.ds(base + j * CH, CH)], didx.at[j], isem)
        for j in range(K)
    ]
    for cp in cps:
        cp.wait()
    plsc.subcore_barrier()

    @pl.loop(0, K)
    def _hist(j):
        pltpu.sync_copy(ones, deg_sh.at[didx.at[j]], add=True)

    plsc.subcore_barrier()
    pltpu.sync_copy(deg_sh.at[pl.ds(s * RZ, RZ)], out_hbm.at[c, pl.ds(s * RZ, RZ)])


@functools.cache
def _sc_degree():
    return pl.kernel(
        _deg_body,
        out_type=jax.ShapeDtypeStruct((NC, NPAD), jnp.float32),
        mesh=_mesh(),
        scratch_types=[
            pltpu.MemorySpace.VMEM_SHARED((NPAD,), jnp.float32),
            pltpu.VMEM((K, CH), jnp.int32),
            pltpu.VMEM((CH,), jnp.float32),
            pltpu.VMEM((RZ,), jnp.float32),
            pltpu.SemaphoreType.DMA,
        ],
    )


def _scat_body(g_hbm, src_hbm, dst_hbm, out_hbm, acc_sh, sidx, didx, rows,
               ssem0, ssem1, ssem2, ssem3, dsem0, dsem1, dsem2, dsem3,
               gsem0, gsem1):
    ssems = (ssem0, ssem1, ssem2, ssem3)
    dsems = (dsem0, dsem1, dsem2, dsem3)
    gsems = (gsem0, gsem1)
    c = lax.axis_index("c")
    s = lax.axis_index("s")
    t = c * NS + s
    base = t * EPT
    row0 = s * RZ

    @pl.loop(0, CH)
    def _zr(r):
        @pl.loop(0, D // 16)
        def _zc(i):
            rows[0, r, pl.ds(i * 16, 16)] = jnp.zeros((16,), jnp.float32)

    for cc in range(RZ // CH):
        pltpu.sync_copy(rows.at[0], acc_sh.at[pl.ds(row0 + cc * CH, CH)])
    plsc.subcore_barrier()

    def idx_load(j, q):
        pltpu.async_copy(src_hbm.at[pl.ds(base + j * CH, CH)], sidx.at[q], ssems[q])
        pltpu.async_copy(dst_hbm.at[pl.ds(base + j * CH, CH)], didx.at[q], dsems[q])

    def idx_wait(q, which):
        hbm = src_hbm if which == "s" else dst_hbm
        buf = sidx if which == "s" else didx
        sem = ssems[q] if which == "s" else dsems[q]
        pltpu.make_async_copy(hbm.at[pl.ds(base, CH)], buf.at[q], sem).wait()

    def gather_start(j, q, bq):
        pltpu.async_copy(g_hbm.at[sidx.at[q]], rows.at[bq], gsems[bq])

    def gather_wait(bq):
        pltpu.make_async_copy(
            g_hbm.at[sidx.at[0]], rows.at[bq], gsems[bq]
        ).wait()

    for q in range(NI):
        idx_load(q, q)
    for bq in range(NB):
        idx_wait(bq, "s")
        gather_start(bq, bq, bq)

    @pl.loop(0, K // NI)
    def _ring(gi):
        for jj in range(NI):
            j = gi * NI + jj
            q = jj
            bq = jj % NB
            gather_wait(bq)
            idx_wait(q, "d")
            pltpu.sync_copy(rows.at[bq], acc_sh.at[didx.at[q]], add=True)

            @pl.when(j + NI < K)
            def _refill():
                idx_load(j + NI, q)

            @pl.when(j + NB < K)
            def _next():
                qn = (jj + NB) % NI
                idx_wait(qn, "s")
                gather_start(j + NB, qn, bq)

    plsc.subcore_barrier()
    for cc in range(RZ // CH):
        pltpu.sync_copy(
            acc_sh.at[pl.ds(row0 + cc * CH, CH)],
            out_hbm.at[c, pl.ds(row0 + cc * CH, CH)],
        )


@functools.cache
def _sc_scatter():
    return pl.kernel(
        _scat_body,
        out_type=jax.ShapeDtypeStruct((NC, NPAD, D), jnp.float32),
        mesh=_mesh(),
        scratch_types=[
            pltpu.MemorySpace.VMEM_SHARED((NPAD, D), jnp.float32),
            pltpu.VMEM((NI, CH), jnp.int32),
            pltpu.VMEM((NI, CH), jnp.int32),
            pltpu.VMEM((NB, CH, D), jnp.float32),
        ] + [pltpu.SemaphoreType.DMA] * (2 * NI + NB),
    )


def _scale_body(x_ref, w_ref, d_ref, g_ref):
    deg = d_ref[0, :] + d_ref[1, :] + 1.0
    dinv = lax.rsqrt(deg)
    h = jnp.dot(x_ref[...], w_ref[...], preferred_element_type=jnp.float32)
    g_ref[...] = h * dinv[:, None]


def _tc_scale(x_p, W, deg2):
    RB = 1024
    return pl.pallas_call(
        _scale_body,
        grid=(NPAD // RB,),
        in_specs=[
            pl.BlockSpec((RB, D), lambda i: (i, 0)),
            pl.BlockSpec((D, D), lambda i: (0, 0)),
            pl.BlockSpec((NC, RB), lambda i: (0, i)),
        ],
        out_specs=pl.BlockSpec((RB, D), lambda i: (i, 0)),
        out_shape=jax.ShapeDtypeStruct((NPAD, D), jnp.float32),
    )(x_p, W, deg2)


def _final_body(x_ref, g_ref, p0_ref, p1_ref, d_ref, b_ref, o_ref):
    deg = d_ref[0, :] + d_ref[1, :] + 1.0
    dinv = lax.rsqrt(deg)
    sacc = p0_ref[...] + p1_ref[...] + g_ref[...]
    conv = sacc * dinv[:, None] + b_ref[...][None, :]
    o_ref[...] = x_ref[...] + jnp.maximum(conv, 0.0)


def _tc_final(x_p, g, p0, p1, deg2, b):
    RB = 1024
    return pl.pallas_call(
        _final_body,
        grid=(NPAD // RB,),
        in_specs=[
            pl.BlockSpec((RB, D), lambda i: (i, 0)),
            pl.BlockSpec((RB, D), lambda i: (i, 0)),
            pl.BlockSpec((RB, D), lambda i: (i, 0)),
            pl.BlockSpec((RB, D), lambda i: (i, 0)),
            pl.BlockSpec((NC, RB), lambda i: (0, i)),
            pl.BlockSpec((D,), lambda i: (0,)),
        ],
        out_specs=pl.BlockSpec((RB, D), lambda i: (i, 0)),
        out_shape=jax.ShapeDtypeStruct((NPAD, D), jnp.float32),
    )(x_p, g, p0, p1, deg2, b)


@jax.jit
def kernel(x, edge_index, W, b):
    src = edge_index[0].astype(jnp.int32)
    dst = edge_index[1].astype(jnp.int32)
    e = src.shape[0]
    npad_e = EPAD - e
    padv = N + (jnp.arange(npad_e, dtype=jnp.int32) % (NPAD - N))
    src_p = jnp.concatenate([src, padv])
    dst_p = jnp.concatenate([dst, padv])
    x_p = jnp.pad(x, ((0, NPAD - N), (0, 0)))

    deg2 = _sc_degree()(dst_p)
    g = _tc_scale(x_p, W, deg2)
    parts = _sc_scatter()(g, src_p, dst_p)
    out = _tc_final(x_p, g, parts[0], parts[1], deg2, b)
    return out[:N]


# speedup vs baseline: 44.3124x; 1.1624x over previous
import functools

import jax
import jax.numpy as jnp
from jax import lax
from jax.experimental import pallas as pl
from jax.experimental.pallas import tpu as pltpu
from jax.experimental.pallas import tpu_sc as plsc

N = 10000
D = 128
E = 320000
NC = 2
NS = 16
NW = NC * NS
NPAD = 10240
RZ = NPAD // NS
EPT = E // NW
CH = 128
K = EPT // CH
TAIL = EPT - K * CH
NB = 2
NI = 2
RB = 1024


@functools.cache
def _mesh():
    return plsc.VectorSubcoreMesh(
        core_axis_name="c", subcore_axis_name="s", num_cores=NC, num_subcores=NS
    )


def _deg_body(ei_hbm, out_hbm, deg_sh, didx, dtail, ones, zbuf, isem):
    c = lax.axis_index("c")
    s = lax.axis_index("s")
    t = c * NS + s
    base = t * EPT

    @pl.loop(0, CH // 16)
    def _fill_ones(i):
        ones[pl.ds(i * 16, 16)] = jnp.full((16,), 1.0, jnp.float32)

    @pl.loop(0, RZ // 16)
    def _fill_zeros(i):
        zbuf[pl.ds(i * 16, 16)] = jnp.zeros((16,), jnp.float32)

    pltpu.sync_copy(zbuf, deg_sh.at[pl.ds(s * RZ, RZ)])

    cps = [
        pltpu.async_copy(ei_hbm.at[pl.ds(E + base + j * CH, CH)], didx.at[j], isem)
        for j in range(K)
    ]
    cps.append(pltpu.async_copy(ei_hbm.at[pl.ds(E + base + K * CH, TAIL)], dtail, isem))
    for cp in cps:
        cp.wait()
    plsc.subcore_barrier()

    @pl.loop(0, K)
    def _hist(j):
        pltpu.sync_copy(ones, deg_sh.at[didx.at[j]], add=True)

    pltpu.sync_copy(ones.at[pl.ds(0, TAIL)], deg_sh.at[dtail], add=True)

    plsc.subcore_barrier()
    pltpu.sync_copy(deg_sh.at[pl.ds(s * RZ, RZ)], out_hbm.at[c, pl.ds(s * RZ, RZ)])


@functools.cache
def _sc_degree():
    return pl.kernel(
        _deg_body,
        out_type=jax.ShapeDtypeStruct((NC, NPAD), jnp.float32),
        mesh=_mesh(),
        scratch_types=[
            pltpu.MemorySpace.VMEM_SHARED((NPAD,), jnp.float32),
            pltpu.VMEM((K, CH), jnp.int32),
            pltpu.VMEM((TAIL,), jnp.int32),
            pltpu.VMEM((CH,), jnp.float32),
            pltpu.VMEM((RZ,), jnp.float32),
            pltpu.SemaphoreType.DMA,
        ],
    )


def _scat_body(g_hbm, ei_hbm, out_hbm, acc_sh, sidx, didx, rows, stail, dtail,
               rtail, ssem0, ssem1, dsem0, dsem1, gsem0, gsem1, tsem):
    ssems = (ssem0, ssem1)
    dsems = (dsem0, dsem1)
    gsems = (gsem0, gsem1)
    c = lax.axis_index("c")
    s = lax.axis_index("s")
    t = c * NS + s
    base = t * EPT
    row0 = s * RZ

    @pl.loop(0, CH)
    def _zr(r):
        @pl.loop(0, D // 16)
        def _zc(i):
            rows[0, r, pl.ds(i * 16, 16)] = jnp.zeros((16,), jnp.float32)

    for cc in range(RZ // CH):
        pltpu.sync_copy(rows.at[0], acc_sh.at[pl.ds(row0 + cc * CH, CH)])
    plsc.subcore_barrier()

    def sidx_load(j, q):
        pltpu.async_copy(ei_hbm.at[pl.ds(base + j * CH, CH)], sidx.at[q], ssems[q])

    def didx_load(j, q):
        pltpu.async_copy(ei_hbm.at[pl.ds(E + base + j * CH, CH)], didx.at[q], dsems[q])

    def sidx_wait(q):
        pltpu.make_async_copy(ei_hbm.at[pl.ds(base, CH)], sidx.at[q], ssems[q]).wait()

    def didx_wait(q):
        pltpu.make_async_copy(ei_hbm.at[pl.ds(E + base, CH)], didx.at[q], dsems[q]).wait()

    def gather_start(q, bq):
        pltpu.async_copy(g_hbm.at[sidx.at[q]], rows.at[bq], gsems[bq])

    def gather_wait(bq):
        pltpu.make_async_copy(g_hbm.at[sidx.at[0]], rows.at[bq], gsems[bq]).wait()

    for q in range(NI):
        sidx_load(q, q)
        didx_load(q, q)
    for bq in range(NB):
        sidx_wait(bq)
        gather_start(bq, bq)

    @pl.loop(0, K // NI)
    def _ring(gi):
        for jj in range(NI):
            j = gi * NI + jj
            q = jj
            gather_wait(q)

            @pl.when(j + NI < K)
            def _refill_s():
                sidx_load(j + NI, q)

            didx_wait(q)
            pltpu.sync_copy(rows.at[q], acc_sh.at[didx.at[q]], add=True)

            @pl.when(j + NI < K)
            def _refill_d():
                didx_load(j + NI, q)

            @pl.when(j + NB < K)
            def _next():
                sidx_wait(q)
                gather_start(q, q)

    pltpu.async_copy(ei_hbm.at[pl.ds(base + K * CH, TAIL)], stail, tsem).wait()
    pltpu.async_copy(ei_hbm.at[pl.ds(E + base + K * CH, TAIL)], dtail, tsem).wait()
    pltpu.async_copy(g_hbm.at[stail], rtail, tsem).wait()
    pltpu.sync_copy(rtail, acc_sh.at[dtail], add=True)

    plsc.subcore_barrier()
    for cc in range(RZ // CH):
        pltpu.sync_copy(
            acc_sh.at[pl.ds(row0 + cc * CH, CH)],
            out_hbm.at[c, pl.ds(row0 + cc * CH, CH)],
        )


@functools.cache
def _sc_scatter():
    return pl.kernel(
        _scat_body,
        out_type=jax.ShapeDtypeStruct((NC, NPAD, D), jnp.float32),
        mesh=_mesh(),
        scratch_types=[
            pltpu.MemorySpace.VMEM_SHARED((NPAD, D), jnp.float32),
            pltpu.VMEM((NI, CH), jnp.int32),
            pltpu.VMEM((NI, CH), jnp.int32),
            pltpu.VMEM((NB, CH, D), jnp.float32),
            pltpu.VMEM((TAIL,), jnp.int32),
            pltpu.VMEM((TAIL,), jnp.int32),
            pltpu.VMEM((TAIL, D), jnp.float32),
        ] + [pltpu.SemaphoreType.DMA] * (2 * NI + NB + 1),
    )


def _scale_body(x_ref, w_ref, d_ref, g_ref):
    deg = d_ref[0, :] + d_ref[1, :] + 1.0
    dinv = lax.rsqrt(deg)
    h = jnp.dot(x_ref[...], w_ref[...], preferred_element_type=jnp.float32)
    g_ref[...] = h * dinv[:, None]


def _tc_scale(x, W, deg2):
    return pl.pallas_call(
        _scale_body,
        grid=(pl.cdiv(N, RB),),
        in_specs=[
            pl.BlockSpec((RB, D), lambda i: (i, 0)),
            pl.BlockSpec((D, D), lambda i: (0, 0)),
            pl.BlockSpec((NC, RB), lambda i: (0, i)),
        ],
        out_specs=pl.BlockSpec((RB, D), lambda i: (i, 0)),
        out_shape=jax.ShapeDtypeStruct((N, D), jnp.float32),
    )(x, W, deg2)


def _final_body(x_ref, g_ref, p_ref, d_ref, b_ref, o_ref):
    deg = d_ref[0, :] + d_ref[1, :] + 1.0
    dinv = lax.rsqrt(deg)
    sacc = p_ref[0] + p_ref[1] + g_ref[...]
    conv = sacc * dinv[:, None] + b_ref[...][None, :]
    o_ref[...] = x_ref[...] + jnp.maximum(conv, 0.0)


def _tc_final(x, g, parts, deg2, b):
    return pl.pallas_call(
        _final_body,
        grid=(pl.cdiv(N, RB),),
        in_specs=[
            pl.BlockSpec((RB, D), lambda i: (i, 0)),
            pl.BlockSpec((RB, D), lambda i: (i, 0)),
            pl.BlockSpec((NC, RB, D), lambda i: (0, i, 0)),
            pl.BlockSpec((NC, RB), lambda i: (0, i)),
            pl.BlockSpec((D,), lambda i: (0,)),
        ],
        out_specs=pl.BlockSpec((RB, D), lambda i: (i, 0)),
        out_shape=jax.ShapeDtypeStruct((N, D), jnp.float32),
    )(x, g, parts, deg2, b)


@jax.jit
def kernel(x, edge_index, W, b):
    ei = edge_index.astype(jnp.int32).reshape(-1)
    deg2 = _sc_degree()(ei)
    g = _tc_scale(x, W, deg2)
    parts = _sc_scatter()(g, ei)
    return _tc_final(x, g, parts, deg2, b)


# speedup vs baseline: 46.4263x; 1.0477x over previous
import functools

import jax
import jax.numpy as jnp
from jax import lax
from jax.experimental import pallas as pl
from jax.experimental.pallas import tpu as pltpu
from jax.experimental.pallas import tpu_sc as plsc

N = 10000
D = 128
E = 320000
NC = 2
NS = 16
NW = NC * NS
NPAD = 10240
RZ = NPAD // NS
CH = 128
KA = E // CH
K = KA // NW
XT = KA - K * NW
NB = 2
NI = 2
ZR = 64
RB = 1024


@functools.cache
def _mesh():
    return plsc.VectorSubcoreMesh(
        core_axis_name="c", subcore_axis_name="s", num_cores=NC, num_subcores=NS
    )


def _deg_body(ei_hbm, out_hbm, deg_sh, didx, ones, zbuf, isem, asem):
    c = lax.axis_index("c")
    s = lax.axis_index("s")
    t = c * NS + s
    base = t * K * CH

    @pl.loop(0, K)
    def _fire_idx(j):
        pltpu.async_copy(
            ei_hbm.at[pl.ds(0, 2), pl.ds(base + j * CH, CH)], didx.at[j], isem
        )

    @pl.loop(0, CH // 16)
    def _fill_ones(i):
        ones[pl.ds(i * 16, 16)] = jnp.full((16,), 1.0, jnp.float32)

    @pl.loop(0, RZ // 16)
    def _fill_zeros(i):
        zbuf[pl.ds(i * 16, 16)] = jnp.zeros((16,), jnp.float32)

    pltpu.sync_copy(zbuf, deg_sh.at[pl.ds(s * RZ, RZ)])

    @pl.loop(0, K)
    def _drain_idx(j):
        pltpu.make_async_copy(
            ei_hbm.at[pl.ds(0, 2), pl.ds(base, CH)], didx.at[0], isem
        ).wait()

    plsc.subcore_barrier()

    @pl.loop(0, K)
    def _fire_hist(j):
        pltpu.async_copy(ones, deg_sh.at[didx.at[j, 1]], asem, add=True)

    @pl.loop(0, K)
    def _drain_hist(j):
        pltpu.make_async_copy(ones, deg_sh.at[didx.at[0, 1]], asem).wait()

    @pl.when(t < XT)
    def _extra():
        pltpu.async_copy(
            ei_hbm.at[pl.ds(0, 2), pl.ds((K * NW + t) * CH, CH)], didx.at[0], isem
        ).wait()
        pltpu.sync_copy(ones, deg_sh.at[didx.at[0, 1]], add=True)

    plsc.subcore_barrier()
    pltpu.sync_copy(deg_sh.at[pl.ds(s * RZ, RZ)], out_hbm.at[c, pl.ds(s * RZ, RZ)])


@functools.cache
def _sc_degree():
    return pl.kernel(
        _deg_body,
        out_type=jax.ShapeDtypeStruct((NC, NPAD), jnp.float32),
        mesh=_mesh(),
        scratch_types=[
            pltpu.MemorySpace.VMEM_SHARED((NPAD,), jnp.float32),
            pltpu.VMEM((K, 2, CH), jnp.int32),
            pltpu.VMEM((CH,), jnp.float32),
            pltpu.VMEM((RZ,), jnp.float32),
            pltpu.SemaphoreType.DMA,
            pltpu.SemaphoreType.DMA,
        ],
    )


def _scat_body(g_hbm, ei_hbm, out_hbm, acc_sh, sib, dib, rows, zbuf,
               ssem0, ssem1, dsem0, dsem1, gsem0, gsem1):
    ssems = (ssem0, ssem1)
    dsems = (dsem0, dsem1)
    gsems = (gsem0, gsem1)
    c = lax.axis_index("c")
    s = lax.axis_index("s")
    t = c * NS + s
    base = t * K * CH
    row0 = s * RZ

    def sidx_load(j, q):
        pltpu.async_copy(
            ei_hbm.at[pl.ds(0, 2), pl.ds(base + j * CH, CH)], sib.at[q], ssems[q]
        )

    def didx_load(j, q):
        pltpu.async_copy(
            ei_hbm.at[pl.ds(0, 2), pl.ds(base + j * CH, CH)], dib.at[q], dsems[q]
        )

    def sidx_wait(q):
        pltpu.make_async_copy(
            ei_hbm.at[pl.ds(0, 2), pl.ds(base, CH)], sib.at[q], ssems[q]
        ).wait()

    def didx_wait(q):
        pltpu.make_async_copy(
            ei_hbm.at[pl.ds(0, 2), pl.ds(base, CH)], dib.at[q], dsems[q]
        ).wait()

    def gather_start(q, bq):
        pltpu.async_copy(g_hbm.at[sib.at[q, 0]], rows.at[bq], gsems[bq])

    def gather_wait(bq):
        pltpu.make_async_copy(g_hbm.at[sib.at[0, 0]], rows.at[bq], gsems[bq]).wait()

    for q in range(NI):
        sidx_load(q, q)
        didx_load(q, q)

    @pl.loop(0, ZR)
    def _zr(r):
        @pl.loop(0, D // 16)
        def _zc(i):
            zbuf[r, pl.ds(i * 16, 16)] = jnp.zeros((16,), jnp.float32)

    for bq in range(NB):
        sidx_wait(bq)
        gather_start(bq, bq)

    for cc in range(RZ // ZR):
        pltpu.sync_copy(zbuf, acc_sh.at[pl.ds(row0 + cc * ZR, ZR)])
    plsc.subcore_barrier()

    @pl.loop(0, K // NI)
    def _ring(gi):
        for jj in range(NI):
            j = gi * NI + jj
            q = jj
            gather_wait(q)

            @pl.when(j + NI < K)
            def _refill_s():
                sidx_load(j + NI, q)

            didx_wait(q)
            pltpu.sync_copy(rows.at[q], acc_sh.at[dib.at[q, 1]], add=True)

            @pl.when(j + NI < K)
            def _refill_d():
                didx_load(j + NI, q)

            @pl.when(j + NB < K)
            def _next():
                sidx_wait(q)
                gather_start(q, q)

    @pl.when(t < XT)
    def _extra():
        pltpu.async_copy(
            ei_hbm.at[pl.ds(0, 2), pl.ds((K * NW + t) * CH, CH)], sib.at[0], ssem0
        ).wait()
        pltpu.async_copy(g_hbm.at[sib.at[0, 0]], rows.at[0], gsem0).wait()
        pltpu.sync_copy(rows.at[0], acc_sh.at[sib.at[0, 1]], add=True)

    plsc.subcore_barrier()
    for cc in range(RZ // CH):
        pltpu.sync_copy(
            acc_sh.at[pl.ds(row0 + cc * CH, CH)],
            out_hbm.at[c, pl.ds(row0 + cc * CH, CH)],
        )


@functools.cache
def _sc_scatter():
    return pl.kernel(
        _scat_body,
        out_type=jax.ShapeDtypeStruct((NC, NPAD, D), jnp.float32),
        mesh=_mesh(),
        scratch_types=[
            pltpu.MemorySpace.VMEM_SHARED((NPAD, D), jnp.float32),
            pltpu.VMEM((NI, 2, CH), jnp.int32),
            pltpu.VMEM((NI, 2, CH), jnp.int32),
            pltpu.VMEM((NB, CH, D), jnp.float32),
            pltpu.VMEM((ZR, D), jnp.float32),
        ] + [pltpu.SemaphoreType.DMA] * (2 * NI + NB),
    )


def _mm_body(x_ref, w_ref, h_ref):
    h_ref[...] = jnp.dot(x_ref[...], w_ref[...], preferred_element_type=jnp.float32)


def _tc_matmul(x, W):
    return pl.pallas_call(
        _mm_body,
        grid=(pl.cdiv(N, RB),),
        in_specs=[
            pl.BlockSpec((RB, D), lambda i: (i, 0)),
            pl.BlockSpec((D, D), lambda i: (0, 0)),
        ],
        out_specs=pl.BlockSpec((RB, D), lambda i: (i, 0)),
        out_shape=jax.ShapeDtypeStruct((N, D), jnp.float32),
    )(x, W)


def _scale_body(h_ref, d_ref, g_ref):
    deg = d_ref[0, :] + d_ref[1, :] + 1.0
    dinv = lax.rsqrt(deg)
    g_ref[...] = h_ref[...] * dinv[:, None]


def _tc_scale(h, deg2):
    return pl.pallas_call(
        _scale_body,
        grid=(pl.cdiv(N, RB),),
        in_specs=[
            pl.BlockSpec((RB, D), lambda i: (i, 0)),
            pl.BlockSpec((NC, RB), lambda i: (0, i)),
        ],
        out_specs=pl.BlockSpec((RB, D), lambda i: (i, 0)),
        out_shape=jax.ShapeDtypeStruct((N, D), jnp.float32),
    )(h, deg2)


def _final_body(x_ref, g_ref, p_ref, d_ref, b_ref, o_ref):
    deg = d_ref[0, :] + d_ref[1, :] + 1.0
    dinv = lax.rsqrt(deg)
    sacc = p_ref[0] + p_ref[1] + g_ref[...]
    conv = sacc * dinv[:, None] + b_ref[...][None, :]
    o_ref[...] = x_ref[...] + jnp.maximum(conv, 0.0)


def _tc_final(x, g, parts, deg2, b):
    return pl.pallas_call(
        _final_body,
        grid=(pl.cdiv(N, RB),),
        in_specs=[
            pl.BlockSpec((RB, D), lambda i: (i, 0)),
            pl.BlockSpec((RB, D), lambda i: (i, 0)),
            pl.BlockSpec((NC, RB, D), lambda i: (0, i, 0)),
            pl.BlockSpec((NC, RB), lambda i: (0, i)),
            pl.BlockSpec((D,), lambda i: (0,)),
        ],
        out_specs=pl.BlockSpec((RB, D), lambda i: (i, 0)),
        out_shape=jax.ShapeDtypeStruct((N, D), jnp.float32),
    )(x, g, parts, deg2, b)


@jax.jit
def kernel(x, edge_index, W, b):
    ei = edge_index.astype(jnp.int32)
    deg2 = _sc_degree()(ei)
    h = _tc_matmul(x, W)
    g = _tc_scale(h, deg2)
    parts = _sc_scatter()(g, ei)
    return _tc_final(x, g, parts, deg2, b)


# speedup vs baseline: 46.6635x; 1.0051x over previous
import functools

import jax
import jax.numpy as jnp
from jax import lax
from jax.experimental import pallas as pl
from jax.experimental.pallas import tpu as pltpu
from jax.experimental.pallas import tpu_sc as plsc

N = 10000
D = 128
E = 320000
NC = 2
NS = 16
NW = NC * NS
NPAD = 10240
RZ = NPAD // NS
CH = 128
KA = E // CH
K = KA // NW
XT = KA - K * NW
NB = 2
NI = 2
ZR = 64
RB = 1024


@functools.cache
def _mesh():
    return plsc.VectorSubcoreMesh(
        core_axis_name="c", subcore_axis_name="s", num_cores=NC, num_subcores=NS
    )


def _deg_body(ei_hbm, out_hbm, deg_sh, didx, ones, zbuf, isem, asem):
    c = lax.axis_index("c")
    s = lax.axis_index("s")
    t = c * NS + s
    base = t * K * CH

    @pl.loop(0, K)
    def _fire_idx(j):
        pltpu.async_copy(
            ei_hbm.at[pl.ds(0, 2), pl.ds(base + j * CH, CH)], didx.at[j], isem
        )

    @pl.loop(0, CH // 16)
    def _fill_ones(i):
        ones[pl.ds(i * 16, 16)] = jnp.full((16,), 1.0, jnp.float32)

    @pl.loop(0, RZ // 16)
    def _fill_zeros(i):
        zbuf[pl.ds(i * 16, 16)] = jnp.zeros((16,), jnp.float32)

    pltpu.sync_copy(zbuf, deg_sh.at[pl.ds(s * RZ, RZ)])

    @pl.loop(0, K)
    def _drain_idx(j):
        pltpu.make_async_copy(
            ei_hbm.at[pl.ds(0, 2), pl.ds(base, CH)], didx.at[0], isem
        ).wait()

    plsc.subcore_barrier()

    @pl.loop(0, K)
    def _fire_hist(j):
        pltpu.async_copy(ones, deg_sh.at[didx.at[j, 1]], asem, add=True)

    @pl.loop(0, K)
    def _drain_hist(j):
        pltpu.make_async_copy(ones, deg_sh.at[didx.at[0, 1]], asem).wait()

    @pl.when(t < XT)
    def _extra():
        pltpu.async_copy(
            ei_hbm.at[pl.ds(0, 2), pl.ds((K * NW + t) * CH, CH)], didx.at[0], isem
        ).wait()
        pltpu.sync_copy(ones, deg_sh.at[didx.at[0, 1]], add=True)

    plsc.subcore_barrier()
    pltpu.sync_copy(deg_sh.at[pl.ds(s * RZ, RZ)], out_hbm.at[c, pl.ds(s * RZ, RZ)])


@functools.cache
def _sc_degree():
    return pl.kernel(
        _deg_body,
        out_type=jax.ShapeDtypeStruct((NC, NPAD), jnp.float32),
        mesh=_mesh(),
        scratch_types=[
            pltpu.MemorySpace.VMEM_SHARED((NPAD,), jnp.float32),
            pltpu.VMEM((K, 2, CH), jnp.int32),
            pltpu.VMEM((CH,), jnp.float32),
            pltpu.VMEM((RZ,), jnp.float32),
            pltpu.SemaphoreType.DMA,
            pltpu.SemaphoreType.DMA,
        ],
    )


def _scat_body(g_hbm, ei_hbm, out_hbm, acc_sh, sib, dib, rows, zbuf,
               ssem0, ssem1, dsem0, dsem1, gsem0, gsem1):
    ssems = (ssem0, ssem1)
    dsems = (dsem0, dsem1)
    gsems = (gsem0, gsem1)
    c = lax.axis_index("c")
    s = lax.axis_index("s")
    t = c * NS + s
    base = t * K * CH
    row0 = s * RZ

    def sidx_load(j, q):
        pltpu.async_copy(
            ei_hbm.at[pl.ds(0, 2), pl.ds(base + j * CH, CH)], sib.at[q], ssems[q]
        )

    def didx_load(j, q):
        pltpu.async_copy(
            ei_hbm.at[pl.ds(0, 2), pl.ds(base + j * CH, CH)], dib.at[q], dsems[q]
        )

    def sidx_wait(q):
        pltpu.make_async_copy(
            ei_hbm.at[pl.ds(0, 2), pl.ds(base, CH)], sib.at[q], ssems[q]
        ).wait()

    def didx_wait(q):
        pltpu.make_async_copy(
            ei_hbm.at[pl.ds(0, 2), pl.ds(base, CH)], dib.at[q], dsems[q]
        ).wait()

    def gather_start(q, bq):
        pltpu.async_copy(g_hbm.at[sib.at[q, 0]], rows.at[bq], gsems[bq])

    def gather_wait(bq):
        pltpu.make_async_copy(g_hbm.at[sib.at[0, 0]], rows.at[bq], gsems[bq]).wait()

    for q in range(NI):
        sidx_load(q, q)
        didx_load(q, q)

    @pl.loop(0, ZR)
    def _zr(r):
        @pl.loop(0, D // 16)
        def _zc(i):
            zbuf[r, pl.ds(i * 16, 16)] = jnp.zeros((16,), jnp.float32)

    for bq in range(NB):
        sidx_wait(bq)
        gather_start(bq, bq)

    for cc in range(RZ // ZR):
        pltpu.sync_copy(zbuf, acc_sh.at[pl.ds(row0 + cc * ZR, ZR)])
    plsc.subcore_barrier()

    @pl.loop(0, K // NI)
    def _ring(gi):
        for jj in range(NI):
            j = gi * NI + jj
            q = jj
            gather_wait(q)

            @pl.when(j + NI < K)
            def _refill_s():
                sidx_load(j + NI, q)

            didx_wait(q)
            pltpu.sync_copy(rows.at[q], acc_sh.at[dib.at[q, 1]], add=True)

            @pl.when(j + NI < K)
            def _refill_d():
                didx_load(j + NI, q)

            @pl.when(j + NB < K)
            def _next():
                sidx_wait(q)
                gather_start(q, q)

    @pl.when(t < XT)
    def _extra():
        pltpu.async_copy(
            ei_hbm.at[pl.ds(0, 2), pl.ds((K * NW + t) * CH, CH)], sib.at[0], ssem0
        ).wait()
        pltpu.async_copy(g_hbm.at[sib.at[0, 0]], rows.at[0], gsem0).wait()
        pltpu.sync_copy(rows.at[0], acc_sh.at[sib.at[0, 1]], add=True)

    plsc.subcore_barrier()
    for cc in range(RZ // CH):
        pltpu.sync_copy(
            acc_sh.at[pl.ds(row0 + cc * CH, CH)],
            out_hbm.at[c, pl.ds(row0 + cc * CH, CH)],
        )


@functools.cache
def _sc_scatter():
    return pl.kernel(
        _scat_body,
        out_type=jax.ShapeDtypeStruct((NC, NPAD, D), jnp.float32),
        mesh=_mesh(),
        scratch_types=[
            pltpu.MemorySpace.VMEM_SHARED((NPAD, D), jnp.float32),
            pltpu.VMEM((NI, 2, CH), jnp.int32),
            pltpu.VMEM((NI, 2, CH), jnp.int32),
            pltpu.VMEM((NB, CH, D), jnp.float32),
            pltpu.VMEM((ZR, D), jnp.float32),
        ] + [pltpu.SemaphoreType.DMA] * (2 * NI + NB),
    )


def _scale_body(x_ref, w_ref, d_ref, g_ref):
    deg = d_ref[0, :] + d_ref[1, :] + 1.0
    dinv = lax.rsqrt(deg)
    h = jnp.dot(x_ref[...], w_ref[...], preferred_element_type=jnp.float32)
    g_ref[...] = h * dinv[:, None]


def _tc_scale(x, W, deg2):
    return pl.pallas_call(
        _scale_body,
        grid=(pl.cdiv(N, RB),),
        in_specs=[
            pl.BlockSpec((RB, D), lambda i: (i, 0)),
            pl.BlockSpec((D, D), lambda i: (0, 0)),
            pl.BlockSpec((NC, RB), lambda i: (0, i)),
        ],
        out_specs=pl.BlockSpec((RB, D), lambda i: (i, 0)),
        out_shape=jax.ShapeDtypeStruct((N, D), jnp.float32),
    )(x, W, deg2)


def _final_body(x_ref, g_ref, p_ref, d_ref, b_ref, o_ref):
    deg = d_ref[0, :] + d_ref[1, :] + 1.0
    dinv = lax.rsqrt(deg)
    sacc = p_ref[0] + p_ref[1] + g_ref[...]
    conv = sacc * dinv[:, None] + b_ref[...][None, :]
    o_ref[...] = x_ref[...] + jnp.maximum(conv, 0.0)


def _tc_final(x, g, parts, deg2, b):
    return pl.pallas_call(
        _final_body,
        grid=(pl.cdiv(N, RB),),
        in_specs=[
            pl.BlockSpec((RB, D), lambda i: (i, 0)),
            pl.BlockSpec((RB, D), lambda i: (i, 0)),
            pl.BlockSpec((NC, RB, D), lambda i: (0, i, 0)),
            pl.BlockSpec((NC, RB), lambda i: (0, i)),
            pl.BlockSpec((D,), lambda i: (0,)),
        ],
        out_specs=pl.BlockSpec((RB, D), lambda i: (i, 0)),
        out_shape=jax.ShapeDtypeStruct((N, D), jnp.float32),
    )(x, g, parts, deg2, b)


@jax.jit
def kernel(x, edge_index, W, b):
    ei = edge_index.astype(jnp.int32)
    deg2 = _sc_degree()(ei)
    g = _tc_scale(x, W, deg2)
    parts = _sc_scatter()(g, ei)
    return _tc_final(x, g, parts, deg2, b)
